# Initial kernel scaffold; baseline (speedup 1.0000x reference)
import jax
import jax.numpy as jnp
from jax.experimental import pallas as pl


def kernel(support_features, query_features, support_labels, query_labels):
    raise NotImplementedError("write your pallas kernel here")


# speedup vs baseline: 3.3964x; 3.3964x over previous
import functools

import jax
import jax.numpy as jnp
from jax import lax
from jax.experimental import pallas as pl
from jax.experimental.pallas import tpu as pltpu
from jax.experimental.pallas import tpu_sc as plsc

N_SUPPORT = 16384
D = 256
NUM_CLASSES = 100
L = 16
NC = 2
NS = 16
ROWS_PER_TILE = N_SUPPORT // NS
SUB = 128
NSUB = ROWS_PER_TILE // SUB
DC = D // NC
CLS_PAD = 128
CPT = CLS_PAD // NS
CLS_ROWS = N_SUPPORT // SUB
FLAT_PAD = N_SUPPORT + L
BSEARCH_STEPS = 15


def _first_ge(flat_v, c):
    def step(_, lohi):
        lo, hi = lohi
        mid = lax.div(lo + hi, jnp.int32(2))
        ge = flat_v[pl.ds(mid, L)][0] >= c
        return (jnp.where(ge, lo, mid + 1), jnp.where(ge, mid, hi))
    lo, _ = lax.fori_loop(
        0, BSEARCH_STEPS, step, (jnp.int32(0), jnp.int32(N_SUPPORT)))
    return lo


def _seg_mean_body(feat_hbm, cls2d_hbm, cls1d_hbm, out_hbm,
                   idx_v, flat_v, buf_v, blk_v, acc_sh):
    cid = lax.axis_index("c")
    sid = lax.axis_index("s")
    col0 = cid * DC
    row0 = sid * ROWS_PER_TILE

    zeros16 = jnp.zeros((L,), jnp.float32)

    pltpu.sync_copy(cls2d_hbm.at[pl.ds(sid * NSUB, NSUB)], idx_v)
    pltpu.sync_copy(cls1d_hbm, flat_v)

    @pl.when(sid == 0)
    def _():
        def zrow(i, carry):
            for k in range(DC // L):
                buf_v[i, pl.ds(k * L, L)] = zeros16
            return carry
        lax.fori_loop(0, CLS_PAD, zrow, 0)
        pltpu.sync_copy(buf_v.at[pl.ds(0, CLS_PAD)], acc_sh)

    plsc.subcore_barrier()

    for j in range(NSUB):
        pltpu.sync_copy(
            feat_hbm.at[pl.ds(row0 + j * SUB, SUB), pl.ds(col0, DC)], buf_v)
        pltpu.sync_copy(buf_v, acc_sh.at[idx_v.at[j]], add=True)

    plsc.subcore_barrier()

    start = sid * CPT
    pltpu.sync_copy(acc_sh.at[pl.ds(start, CPT)], blk_v)
    bound = _first_ge(flat_v, start)
    for i in range(CPT):
        nxt = _first_ge(flat_v, start + (i + 1))
        cnt = jnp.maximum(nxt - bound, 1).astype(jnp.float32)
        inv = jnp.full((L,), cnt, jnp.float32)
        for k in range(DC // L):
            blk_v[i, pl.ds(k * L, L)] = blk_v[i, pl.ds(k * L, L)] / inv
        bound = nxt
    pltpu.sync_copy(blk_v, out_hbm.at[pl.ds(start, CPT), pl.ds(col0, DC)])


@jax.jit
def _seg_mean(support_features, cls2d, cls1d):
    mesh = plsc.VectorSubcoreMesh(core_axis_name="c", subcore_axis_name="s")
    run = functools.partial(
        pl.kernel,
        out_type=jax.ShapeDtypeStruct((CLS_PAD, D), jnp.float32),
        mesh=mesh,
        scratch_types=[
            pltpu.VMEM((NSUB, SUB), jnp.int32),
            pltpu.VMEM((FLAT_PAD,), jnp.int32),
            pltpu.VMEM((SUB, DC), jnp.float32),
            pltpu.VMEM((CPT, DC), jnp.float32),
            pltpu.VMEM_SHARED((CLS_PAD, DC), jnp.float32),
        ],
    )(_seg_mean_body)
    padded = run(support_features, cls2d, cls1d)
    return padded[:NUM_CLASSES]


def kernel(support_features, query_features, support_labels, query_labels):
    cls = support_labels[:, 0]
    cls2d = cls.reshape(CLS_ROWS, SUB)
    cls1d = jnp.pad(cls, (0, L), constant_values=NUM_CLASSES)
    return _seg_mean(support_features, cls2d, cls1d)


# speedup vs baseline: 3.9740x; 1.1700x over previous
import functools

import jax
import jax.numpy as jnp
from jax import lax
from jax.experimental import pallas as pl
from jax.experimental.pallas import tpu as pltpu
from jax.experimental.pallas import tpu_sc as plsc

N_SUPPORT = 16384
D = 256
NUM_CLASSES = 100
L = 16
NC = 2
NS = 16
ROWS_PER_TILE = N_SUPPORT // NS
SUB = 128
NSUB = ROWS_PER_TILE // SUB
DC = D // NC
CLS_PAD = 128
CPT = CLS_PAD // NS
CLS_ROWS = N_SUPPORT // SUB
FLAT_PAD = N_SUPPORT + L
BSEARCH_STEPS = 15


def _first_ge(flat_v, c):
    def step(_, lohi):
        lo, hi = lohi
        mid = lax.div(lo + hi, jnp.int32(2))
        ge = flat_v[pl.ds(mid, L)][0] >= c
        return (jnp.where(ge, lo, mid + 1), jnp.where(ge, mid, hi))
    lo, _ = lax.fori_loop(
        0, BSEARCH_STEPS, step, (jnp.int32(0), jnp.int32(N_SUPPORT)))
    return lo


def _seg_mean_body(feat_hbm, cls2d_hbm, cls1d_hbm, out_hbm,
                   idx_v, flat_v, buf0_v, buf1_v, blk_v, acc_sh,
                   sem_f, sem_a, sem_b):
    cid = lax.axis_index("c")
    sid = lax.axis_index("s")
    col0 = cid * DC
    row0 = sid * ROWS_PER_TILE

    zeros16 = jnp.zeros((L,), jnp.float32)

    h_flat = pltpu.async_copy(cls1d_hbm, flat_v, sem_f)

    pltpu.sync_copy(cls2d_hbm.at[pl.ds(sid * NSUB, NSUB)], idx_v)

    for i in range(CPT):
        for k in range(DC // L):
            blk_v[i, pl.ds(k * L, L)] = zeros16
    pltpu.sync_copy(blk_v, acc_sh.at[pl.ds(sid * CPT, CPT)])

    bufs = [buf0_v, buf1_v]
    sems = [sem_a, sem_b]

    def start_load(j, b):
        return pltpu.async_copy(
            feat_hbm.at[pl.ds(row0 + j * SUB, SUB), pl.ds(col0, DC)],
            bufs[b], sems[b])

    pending = [start_load(0, 0), None]
    plsc.subcore_barrier()

    for j in range(NSUB):
        b = j & 1
        pending[b].wait()
        if j + 1 < NSUB:
            pending[1 - b] = start_load(j + 1, 1 - b)
        pltpu.sync_copy(bufs[b], acc_sh.at[idx_v.at[j]], add=True)

    plsc.subcore_barrier()
    h_flat.wait()

    start = sid * CPT
    pltpu.sync_copy(acc_sh.at[pl.ds(start, CPT)], blk_v)
    bound = _first_ge(flat_v, start)
    for i in range(CPT):
        nxt = _first_ge(flat_v, start + (i + 1))
        cnt = jnp.maximum(nxt - bound, 1).astype(jnp.float32)
        inv = jnp.full((L,), cnt, jnp.float32)
        for k in range(DC // L):
            blk_v[i, pl.ds(k * L, L)] = blk_v[i, pl.ds(k * L, L)] / inv
        bound = nxt
    pltpu.sync_copy(blk_v, out_hbm.at[pl.ds(start, CPT), pl.ds(col0, DC)])


@jax.jit
def _seg_mean(support_features, cls2d, cls1d):
    mesh = plsc.VectorSubcoreMesh(core_axis_name="c", subcore_axis_name="s")
    run = functools.partial(
        pl.kernel,
        out_type=jax.ShapeDtypeStruct((CLS_PAD, D), jnp.float32),
        mesh=mesh,
        scratch_types=[
            pltpu.VMEM((NSUB, SUB), jnp.int32),
            pltpu.VMEM((FLAT_PAD,), jnp.int32),
            pltpu.VMEM((SUB, DC), jnp.float32),
            pltpu.VMEM((SUB, DC), jnp.float32),
            pltpu.VMEM((CPT, DC), jnp.float32),
            pltpu.VMEM_SHARED((CLS_PAD, DC), jnp.float32),
            pltpu.SemaphoreType.DMA,
            pltpu.SemaphoreType.DMA,
            pltpu.SemaphoreType.DMA,
        ],
    )(_seg_mean_body)
    padded = run(support_features, cls2d, cls1d)
    return padded[:NUM_CLASSES]


def kernel(support_features, query_features, support_labels, query_labels):
    cls = support_labels[:, 0]
    cls2d = cls.reshape(CLS_ROWS, SUB)
    cls1d = jnp.pad(cls, (0, L), constant_values=NUM_CLASSES)
    return _seg_mean(support_features, cls2d, cls1d)


# speedup vs baseline: 4.0532x; 1.0199x over previous
import functools

import jax
import jax.numpy as jnp
from jax import lax
from jax.experimental import pallas as pl
from jax.experimental.pallas import tpu as pltpu
from jax.experimental.pallas import tpu_sc as plsc

N_SUPPORT = 16384
D = 256
NUM_CLASSES = 100
L = 16
NC = 2
NS = 16
ROWS_PER_TILE = N_SUPPORT // NS
SUB = 128
NSUB = ROWS_PER_TILE // SUB
DC = D // NC
CLS_PAD = 128
CPT = CLS_PAD // NS
CLS_ROWS = N_SUPPORT // SUB
FLAT_PAD = N_SUPPORT + L
BSEARCH_STEPS = 15


def _first_ge(flat_v, c):
    def step(_, lohi):
        lo, hi = lohi
        mid = lax.div(lo + hi, jnp.int32(2))
        ge = flat_v[pl.ds(mid, L)][0] >= c
        return (jnp.where(ge, lo, mid + 1), jnp.where(ge, mid, hi))
    lo, _ = lax.fori_loop(
        0, BSEARCH_STEPS, step, (jnp.int32(0), jnp.int32(N_SUPPORT)))
    return lo


def _seg_mean_body(feat_hbm, cls2d_hbm, cls1d_hbm, out_hbm,
                   idx_v, flat_v, buf0_v, buf1_v, blk_v, acc_sh,
                   sem_f, sem_a, sem_b):
    cid = lax.axis_index("c")
    sid = lax.axis_index("s")
    col0 = cid * DC
    row0 = sid * ROWS_PER_TILE

    zeros16 = jnp.zeros((L,), jnp.float32)

    h_flat = pltpu.async_copy(cls1d_hbm, flat_v, sem_f)

    pltpu.sync_copy(cls2d_hbm.at[pl.ds(sid * NSUB, NSUB)], idx_v)

    def zrow(i, carry):
        for k in range(DC // L):
            blk_v[i, pl.ds(k * L, L)] = zeros16
        return carry
    lax.fori_loop(0, CPT, zrow, 0)
    pltpu.sync_copy(blk_v, acc_sh.at[pl.ds(sid * CPT, CPT)])

    def load_slice(j):
        return feat_hbm.at[pl.ds(row0 + j * SUB, SUB), pl.ds(col0, DC)]

    def start_load(j, buf, sem):
        return pltpu.async_copy(load_slice(j), buf, sem)

    def wait_load(j, buf, sem):
        pltpu.make_async_copy(load_slice(j), buf, sem).wait()

    start_load(0, buf0_v, sem_a)
    plsc.subcore_barrier()

    def pipe(i, carry):
        j0 = 2 * i
        wait_load(j0, buf0_v, sem_a)
        start_load(j0 + 1, buf1_v, sem_b)
        pltpu.sync_copy(buf0_v, acc_sh.at[idx_v.at[j0]], add=True)
        wait_load(j0 + 1, buf1_v, sem_b)

        @pl.when(i < NSUB // 2 - 1)
        def _():
            start_load(j0 + 2, buf0_v, sem_a)
        pltpu.sync_copy(buf1_v, acc_sh.at[idx_v.at[j0 + 1]], add=True)
        return carry
    lax.fori_loop(0, NSUB // 2, pipe, 0)

    plsc.subcore_barrier()
    h_flat.wait()

    start = sid * CPT
    pltpu.sync_copy(acc_sh.at[pl.ds(start, CPT)], blk_v)

    def div_row(i, bound):
        nxt = _first_ge(flat_v, start + (i + 1))
        cnt = jnp.maximum(nxt - bound, 1).astype(jnp.float32)
        inv = jnp.full((L,), cnt, jnp.float32)
        for k in range(DC // L):
            blk_v[i, pl.ds(k * L, L)] = blk_v[i, pl.ds(k * L, L)] / inv
        return nxt
    lax.fori_loop(0, CPT, div_row, _first_ge(flat_v, start))
    pltpu.sync_copy(blk_v, out_hbm.at[pl.ds(start, CPT), pl.ds(col0, DC)])


@jax.jit
def _seg_mean(support_features, cls2d, cls1d):
    mesh = plsc.VectorSubcoreMesh(core_axis_name="c", subcore_axis_name="s")
    run = functools.partial(
        pl.kernel,
        out_type=jax.ShapeDtypeStruct((CLS_PAD, D), jnp.float32),
        mesh=mesh,
        scratch_types=[
            pltpu.VMEM((NSUB, SUB), jnp.int32),
            pltpu.VMEM((FLAT_PAD,), jnp.int32),
            pltpu.VMEM((SUB, DC), jnp.float32),
            pltpu.VMEM((SUB, DC), jnp.float32),
            pltpu.VMEM((CPT, DC), jnp.float32),
            pltpu.VMEM_SHARED((CLS_PAD, DC), jnp.float32),
            pltpu.SemaphoreType.DMA,
            pltpu.SemaphoreType.DMA,
            pltpu.SemaphoreType.DMA,
        ],
    )(_seg_mean_body)
    padded = run(support_features, cls2d, cls1d)
    return padded[:NUM_CLASSES]


def kernel(support_features, query_features, support_labels, query_labels):
    cls = support_labels[:, 0]
    cls2d = cls.reshape(CLS_ROWS, SUB)
    cls1d = jnp.pad(cls, (0, L), constant_values=NUM_CLASSES)
    return _seg_mean(support_features, cls2d, cls1d)
